# baseline (speedup 1.0000x reference)
import functools

import jax
import jax.numpy as jnp
from jax import lax
from jax.experimental import pallas as pl
from jax.experimental.pallas import tpu as pltpu
from jax.experimental.pallas import tpu_sc as plsc

B, T, D, K = 16, 2048, 256, 1024
COMMITMENT_WEIGHT = 1.0
M = 512
BT = B * T
NSEG = 4
SEG = BT // NSEG
SEG_NBLK = SEG // M

NW = 32
B_PER_W = SEG // NW
CHUNK = 128
NCHUNK = B_PER_W // CHUNK


def _vq_tc_kernel(z_ref, z2_ref, mask_ref, cb_ref, c2_ref, idx_ref, loss_ref):
    i = pl.program_id(0)

    z_blk = z_ref[...]
    cb = cb_ref[...]
    z2 = z2_ref[...]
    c2 = c2_ref[...]

    dots = jax.lax.dot_general(
        z_blk, cb,
        dimension_numbers=(((1,), (1,)), ((), ())),
        preferred_element_type=jnp.float32,
    )
    dist = z2 - 2.0 * dots + c2
    mins = jnp.min(dist, axis=1, keepdims=True)
    kiota = jax.lax.broadcasted_iota(jnp.int32, (M, K), 1)
    idx = jnp.min(jnp.where(dist == mins, kiota, K), axis=1).astype(jnp.int32)
    idx_ref[0, 0, :] = idx

    maskf = mask_ref[0, 0, :]
    part = jnp.sum(mins[:, 0] * maskf)

    @pl.when(i == 0)
    def _():
        loss_ref[...] = jnp.zeros_like(loss_ref)
    loss_ref[...] = loss_ref[...] + part


def _tc_segment(zf, z2, maskf, codebook, c2):
    return pl.pallas_call(
        _vq_tc_kernel,
        grid=(SEG_NBLK,),
        in_specs=[
            pl.BlockSpec((M, D), lambda i: (i, 0)),
            pl.BlockSpec((M, 1), lambda i: (i, 0)),
            pl.BlockSpec((1, 1, M), lambda i: (i, 0, 0)),
            pl.BlockSpec((K, D), lambda i: (0, 0)),
            pl.BlockSpec((1, K), lambda i: (0, 0)),
        ],
        out_specs=[
            pl.BlockSpec((1, 1, M), lambda i: (i, 0, 0)),
            pl.BlockSpec((1, 1), lambda i: (0, 0)),
        ],
        out_shape=[
            jax.ShapeDtypeStruct((SEG_NBLK, 1, M), jnp.int32),
            jax.ShapeDtypeStruct((1, 1), jnp.float32),
        ],
    )(zf, z2, maskf, codebook, c2)


def _gather_sc_body(cb_hbm, idx_hbm, out_hbm, idx_v, rows_v, sems):
    wid = lax.axis_index("s") * 2 + lax.axis_index("c")
    base = wid * B_PER_W
    pltpu.sync_copy(idx_hbm.at[pl.ds(base, B_PER_W)], idx_v)

    def gather(c, slot):
        return pltpu.async_copy(
            cb_hbm.at[idx_v.at[pl.ds(c * CHUNK, CHUNK)]],
            rows_v.at[slot], sems.at[slot])

    cp0 = gather(0, 0)
    for c in range(NCHUNK):
        cp = cp0 if c == 0 else cpn
        if c + 1 < NCHUNK:
            cpn = gather(c + 1, (c + 1) % 2)
        cp.wait()
        pltpu.sync_copy(rows_v.at[c % 2],
                        out_hbm.at[pl.ds(base + c * CHUNK, CHUNK)])


def _sc_gather(codebook, idx_flat):
    mesh = plsc.VectorSubcoreMesh(core_axis_name="c", subcore_axis_name="s")
    return pl.kernel(
        _gather_sc_body,
        out_type=jax.ShapeDtypeStruct((SEG, D), jnp.float32),
        mesh=mesh,
        scratch_types=[
            pltpu.VMEM((B_PER_W,), jnp.int32),
            pltpu.VMEM((2, CHUNK, D), jnp.float32),
            pltpu.SemaphoreType.DMA((2,)),
        ],
    )(codebook, idx_flat)


@jax.jit
def kernel(z, mask, codebook):
    zf = z.reshape(BT, D)
    z2 = jnp.sum(z * z, axis=-1, keepdims=True).reshape(BT, 1)
    c2 = jnp.sum(codebook * codebook, axis=-1).reshape(1, K)
    maskf = mask.astype(jnp.float32).reshape(BT // M, 1, M)

    idx_segs, q_segs, loss_parts = [], [], []
    for s in range(NSEG):
        idx_s, loss_s = _tc_segment(
            lax.slice_in_dim(zf, s * SEG, (s + 1) * SEG, axis=0),
            lax.slice_in_dim(z2, s * SEG, (s + 1) * SEG, axis=0),
            lax.slice_in_dim(maskf, s * SEG_NBLK, (s + 1) * SEG_NBLK, axis=0),
            codebook, c2)
        idx_segs.append(idx_s)
        loss_parts.append(loss_s[0, 0])
        q_segs.append(_sc_gather(codebook, idx_s.reshape(SEG)))

    idx = jnp.concatenate(idx_segs, axis=0)
    q = jnp.concatenate(q_segs, axis=0)

    quantized = q.reshape(B, T, D)
    indices = idx.reshape(B, T)
    denom = jnp.maximum(jnp.sum(mask.astype(jnp.float32)), 1.0) * D
    loss_sum = loss_parts[0]
    for p in loss_parts[1:]:
        loss_sum = loss_sum + p
    commit_loss = (loss_sum / denom) * COMMITMENT_WEIGHT

    quantized_st = z + jax.lax.stop_gradient(quantized - z)
    return quantized_st, indices, commit_loss


# speedup vs baseline: 1.3885x; 1.3885x over previous
import functools

import jax
import jax.numpy as jnp
from jax import lax
from jax.experimental import pallas as pl
from jax.experimental.pallas import tpu as pltpu
from jax.experimental.pallas import tpu_sc as plsc

B, T, D, K = 16, 2048, 256, 1024
COMMITMENT_WEIGHT = 1.0
M = 512
BT = B * T
NBLK = BT // M

NW = 32
B_PER_W = BT // NW
CHUNK = 128
NCHUNK = B_PER_W // CHUNK


def _vq_tc_kernel(z_ref, z2_ref, mask_ref, cb_ref, c2_ref, idx_ref, loss_ref):
    i = pl.program_id(0)

    z_blk = z_ref[...]
    cb = cb_ref[...]
    z2 = z2_ref[...]
    c2 = c2_ref[...]

    dots = jax.lax.dot_general(
        z_blk, cb,
        dimension_numbers=(((1,), (1,)), ((), ())),
        preferred_element_type=jnp.float32,
    )
    dist = z2 - 2.0 * dots + c2
    mins = jnp.min(dist, axis=1, keepdims=True)
    kiota = jax.lax.broadcasted_iota(jnp.int32, (M, K), 1)
    idx = jnp.min(jnp.where(dist == mins, kiota, K), axis=1).astype(jnp.int32)
    idx_ref[0, 0, :] = idx

    maskf = mask_ref[0, 0, :]
    part = jnp.sum(mins[:, 0] * maskf)

    @pl.when(i == 0)
    def _():
        loss_ref[...] = jnp.zeros_like(loss_ref)
    loss_ref[...] = loss_ref[...] + part


def _gather_sc_body(cb_hbm, idx_hbm, out_hbm, idx_v, rows_v, sems):
    wid = lax.axis_index("s") * 2 + lax.axis_index("c")
    base = wid * B_PER_W
    pltpu.sync_copy(idx_hbm.at[pl.ds(base, B_PER_W)], idx_v)

    def gather(c, slot):
        return pltpu.async_copy(
            cb_hbm.at[idx_v.at[pl.ds(c * CHUNK, CHUNK)]],
            rows_v.at[slot], sems.at[slot])

    cp0 = gather(0, 0)
    for c in range(NCHUNK):
        cp = cp0 if c == 0 else cpn
        if c + 1 < NCHUNK:
            cpn = gather(c + 1, (c + 1) % 2)
        cp.wait()
        pltpu.sync_copy(rows_v.at[c % 2],
                        out_hbm.at[pl.ds(base + c * CHUNK, CHUNK)])


def _sc_gather(codebook, idx_flat):
    mesh = plsc.VectorSubcoreMesh(core_axis_name="c", subcore_axis_name="s")
    return pl.kernel(
        _gather_sc_body,
        out_type=jax.ShapeDtypeStruct((BT, D), jnp.float32),
        mesh=mesh,
        scratch_types=[
            pltpu.VMEM((B_PER_W,), jnp.int32),
            pltpu.VMEM((2, CHUNK, D), jnp.float32),
            pltpu.SemaphoreType.DMA((2,)),
        ],
    )(codebook, idx_flat)


@jax.jit
def kernel(z, mask, codebook):
    zf = z.reshape(BT, D)
    z2 = jnp.sum(z * z, axis=-1, keepdims=True).reshape(BT, 1)
    c2 = jnp.sum(codebook * codebook, axis=-1).reshape(1, K)
    maskf = mask.astype(jnp.float32).reshape(NBLK, 1, M)

    idx, loss_sum = pl.pallas_call(
        _vq_tc_kernel,
        grid=(NBLK,),
        in_specs=[
            pl.BlockSpec((M, D), lambda i: (i, 0)),
            pl.BlockSpec((M, 1), lambda i: (i, 0)),
            pl.BlockSpec((1, 1, M), lambda i: (i, 0, 0)),
            pl.BlockSpec((K, D), lambda i: (0, 0)),
            pl.BlockSpec((1, K), lambda i: (0, 0)),
        ],
        out_specs=[
            pl.BlockSpec((1, 1, M), lambda i: (i, 0, 0)),
            pl.BlockSpec((1, 1), lambda i: (0, 0)),
        ],
        out_shape=[
            jax.ShapeDtypeStruct((NBLK, 1, M), jnp.int32),
            jax.ShapeDtypeStruct((1, 1), jnp.float32),
        ],
    )(zf, z2, maskf, codebook, c2)

    q = _sc_gather(codebook, idx.reshape(BT))

    quantized_st = q.reshape(B, T, D)
    indices = idx.reshape(B, T)
    denom = jnp.maximum(jnp.sum(mask.astype(jnp.float32)), 1.0) * D
    commit_loss = (loss_sum[0, 0] / denom) * COMMITMENT_WEIGHT

    return quantized_st, indices, commit_loss


# speedup vs baseline: 1.8659x; 1.3438x over previous
import functools

import jax
import jax.numpy as jnp
from jax import lax
from jax.experimental import pallas as pl
from jax.experimental.pallas import tpu as pltpu
from jax.experimental.pallas import tpu_sc as plsc

B, T, D, K = 16, 2048, 256, 1024
COMMITMENT_WEIGHT = 1.0
M = 8192
BT = B * T
NBLK = BT // M

NW = 32
B_PER_W = BT // NW
CHUNK = 128
NCHUNK = B_PER_W // CHUNK


def _vq_tc_kernel(z_ref, z2_ref, mask_ref, cb_ref, c2_ref, idx_ref, loss_ref):
    i = pl.program_id(0)

    z_blk = z_ref[...]
    cb = cb_ref[...]
    z2 = z2_ref[...]
    c2 = c2_ref[...]

    dots = jax.lax.dot_general(
        z_blk, cb,
        dimension_numbers=(((1,), (1,)), ((), ())),
        preferred_element_type=jnp.float32,
    )
    dist = z2 - 2.0 * dots + c2
    mins = jnp.min(dist, axis=1, keepdims=True)
    kiota = jax.lax.broadcasted_iota(jnp.int32, (M, K), 1)
    idx = jnp.min(jnp.where(dist == mins, kiota, K), axis=1).astype(jnp.int32)
    idx_ref[0, 0, :] = idx

    maskf = mask_ref[0, 0, :]
    part = jnp.sum(mins[:, 0] * maskf)

    @pl.when(i == 0)
    def _():
        loss_ref[...] = jnp.zeros_like(loss_ref)
    loss_ref[...] = loss_ref[...] + part


def _gather_sc_body(cb_hbm, idx_hbm, out_hbm, idx_v, rows_v, sems):
    wid = lax.axis_index("s") * 2 + lax.axis_index("c")
    base = wid * B_PER_W
    pltpu.sync_copy(idx_hbm.at[pl.ds(base, B_PER_W)], idx_v)

    def gather(c, slot):
        return pltpu.async_copy(
            cb_hbm.at[idx_v.at[pl.ds(c * CHUNK, CHUNK)]],
            rows_v.at[slot], sems.at[slot])

    cps = [gather(0, 0), gather(1, 1)]
    for c in range(NCHUNK):
        if c + 2 < NCHUNK:
            cps.append(gather(c + 2, (c + 2) % 3))
        cps[c].wait()
        pltpu.sync_copy(rows_v.at[c % 3],
                        out_hbm.at[pl.ds(base + c * CHUNK, CHUNK)])


def _sc_gather(codebook, idx_flat):
    mesh = plsc.VectorSubcoreMesh(core_axis_name="c", subcore_axis_name="s")
    return pl.kernel(
        _gather_sc_body,
        out_type=jax.ShapeDtypeStruct((BT, D), jnp.float32),
        mesh=mesh,
        scratch_types=[
            pltpu.VMEM((B_PER_W,), jnp.int32),
            pltpu.VMEM((3, CHUNK, D), jnp.float32),
            pltpu.SemaphoreType.DMA((3,)),
        ],
    )(codebook, idx_flat)


@jax.jit
def kernel(z, mask, codebook):
    zf = z.reshape(BT, D)
    z2 = jnp.sum(z * z, axis=-1, keepdims=True).reshape(BT, 1)
    c2 = jnp.sum(codebook * codebook, axis=-1).reshape(1, K)
    maskf = mask.astype(jnp.float32).reshape(NBLK, 1, M)

    idx, loss_sum = pl.pallas_call(
        _vq_tc_kernel,
        grid=(NBLK,),
        in_specs=[
            pl.BlockSpec((M, D), lambda i: (i, 0)),
            pl.BlockSpec((M, 1), lambda i: (i, 0)),
            pl.BlockSpec((1, 1, M), lambda i: (i, 0, 0)),
            pl.BlockSpec((K, D), lambda i: (0, 0)),
            pl.BlockSpec((1, K), lambda i: (0, 0)),
        ],
        out_specs=[
            pl.BlockSpec((1, 1, M), lambda i: (i, 0, 0)),
            pl.BlockSpec((1, 1), lambda i: (0, 0)),
        ],
        out_shape=[
            jax.ShapeDtypeStruct((NBLK, 1, M), jnp.int32),
            jax.ShapeDtypeStruct((1, 1), jnp.float32),
        ],
    )(zf, z2, maskf, codebook, c2)

    q = _sc_gather(codebook, idx.reshape(BT))

    quantized_st = q.reshape(B, T, D)
    indices = idx.reshape(B, T)
    denom = jnp.maximum(jnp.sum(mask.astype(jnp.float32)), 1.0) * D
    commit_loss = (loss_sum[0, 0] / denom) * COMMITMENT_WEIGHT

    return quantized_st, indices, commit_loss
